# baseline (speedup 1.0000x reference)
import jax
import jax.numpy as jnp
from jax import lax
from jax.experimental import pallas as pl
from jax.experimental.pallas import tpu as pltpu
from jax.experimental.pallas import tpu_sc as plsc

N_NODES = 10000
N_EDGES = 320000
D = 128
NEG = 10
INS_LOSS_W = 1e-05

NC = 2
NS = 16
NW = NC * NS
ACC_ROWS = 10240
STRIPE = ACC_ROWS // NS
EK = 128
IB = 16
IDX_ROWS = 2560
E_PAD = IDX_ROWS * EK
NBLK = IDX_ROWS // IB


def _sc_segsum_body(y_hbm, src_hbm, dst_hbm, zeros_hbm, out_hbm,
                    acc_shared, rows_a, rows_b, src_blk, dst_blk,
                    src_blk2, dst_blk2,
                    gsem_a, gsem_b, ssem_a, ssem_b, isem_s, isem_d):
    c = lax.axis_index("c")
    s = lax.axis_index("s")
    w = c * NS + s

    row0 = s * STRIPE
    pi0 = pltpu.async_copy(src_hbm.at[pl.ds(w * IB, IB), :], src_blk, isem_s)
    pi1 = pltpu.async_copy(dst_hbm.at[pl.ds(w * IB, IB), :], dst_blk, isem_d)
    pltpu.sync_copy(zeros_hbm, acc_shared.at[pl.ds(row0, STRIPE), :])
    plsc.subcore_barrier()
    pi0.wait()
    pi1.wait()

    n_iter = NBLK // NW
    isets = ((src_blk, dst_blk), (src_blk2, dst_blk2))

    def _process(i, cur_set, nxt_set):
        src_c, dst_c = cur_set
        src_n, dst_n = nxt_set
        idesc = []

        @pl.when(i < n_iter - 1)
        def _prefetch():
            r1 = (w + (i + 1) * NW) * IB
            idesc.append(pltpu.async_copy(
                src_hbm.at[pl.ds(r1, IB), :], src_n, isem_s))
            idesc.append(pltpu.async_copy(
                dst_hbm.at[pl.ds(r1, IB), :], dst_n, isem_d))

        bufs = ((rows_a, gsem_a, ssem_a), (rows_b, gsem_b, ssem_b))
        gd = [None, None]
        sd = [None, None]
        gd[0] = pltpu.async_copy(y_hbm.at[src_c.at[0]], rows_a, gsem_a)
        for j in range(IB):
            cur = j % 2
            nxt = (j + 1) % 2
            buf, _, ssem = bufs[cur]
            nbuf, ngsem, _ = bufs[nxt]
            if j + 1 < IB:
                if sd[nxt] is not None:
                    sd[nxt].wait()
                gd[nxt] = pltpu.async_copy(
                    y_hbm.at[src_c.at[j + 1]], nbuf, ngsem)
            gd[cur].wait()
            sd[cur] = pltpu.async_copy(
                buf, acc_shared.at[dst_c.at[j]], ssem, add=True)
        sd[0].wait()
        sd[1].wait()

        @pl.when(i < n_iter - 1)
        def _wait_prefetch():
            pltpu.make_async_copy(
                src_hbm.at[pl.ds(0, IB), :], src_n, isem_s).wait()
            pltpu.make_async_copy(
                dst_hbm.at[pl.ds(0, IB), :], dst_n, isem_d).wait()

    def _block(i, _):
        @pl.when(i % 2 == 0)
        def _even():
            _process(i, isets[0], isets[1])

        @pl.when(i % 2 == 1)
        def _odd():
            _process(i, isets[1], isets[0])
        return ()
    lax.fori_loop(0, n_iter, _block, ())
    plsc.subcore_barrier()

    @pl.when(s < NS - 1)
    def _wr_full():
        pltpu.sync_copy(acc_shared.at[pl.ds(row0, STRIPE), :],
                        out_hbm.at[pl.ds(c * N_NODES + row0, STRIPE), :])

    @pl.when(s == NS - 1)
    def _wr_tail():
        tail = N_NODES - (NS - 1) * STRIPE
        base = (NS - 1) * STRIPE
        pltpu.sync_copy(acc_shared.at[pl.ds(base, tail), :],
                        out_hbm.at[pl.ds(c * N_NODES + base, tail), :])


def _sc_segsum(y, src2d, dst2d, zeros_stripe):
    mesh = plsc.VectorSubcoreMesh(core_axis_name="c", subcore_axis_name="s")
    return pl.kernel(
        _sc_segsum_body,
        out_type=jax.ShapeDtypeStruct((2 * N_NODES, D), jnp.float32),
        mesh=mesh,
        scratch_types=[
            pltpu.VMEM_SHARED((ACC_ROWS, D), jnp.float32),
            pltpu.VMEM((EK, D), jnp.float32),
            pltpu.VMEM((EK, D), jnp.float32),
            pltpu.VMEM((IB, EK), jnp.int32),
            pltpu.VMEM((IB, EK), jnp.int32),
            pltpu.VMEM((IB, EK), jnp.int32),
            pltpu.VMEM((IB, EK), jnp.int32),
            pltpu.SemaphoreType.DMA,
            pltpu.SemaphoreType.DMA,
            pltpu.SemaphoreType.DMA,
            pltpu.SemaphoreType.DMA,
            pltpu.SemaphoreType.DMA,
            pltpu.SemaphoreType.DMA,
        ],
    )(y, src2d, dst2d, zeros_stripe)


RB = 1000
NB_G = N_NODES // RB


def _mm_body(x_ref, w_ref, o_ref):
    o_ref[...] = jnp.dot(x_ref[...], w_ref[...],
                         preferred_element_type=jnp.float32)


def _matmul(x, w):
    return pl.pallas_call(
        _mm_body,
        grid=(NB_G,),
        in_specs=[pl.BlockSpec((RB, D), lambda i: (i, 0)),
                  pl.BlockSpec((D, D), lambda i: (0, 0))],
        out_specs=pl.BlockSpec((RB, D), lambda i: (i, 0)),
        out_shape=jax.ShapeDtypeStruct((N_NODES, D), jnp.float32),
    )(x, w)


def _relu_mm_body(p0_ref, p1_ref, b_ref, w_ref, o_ref):
    h = jnp.maximum(p0_ref[...] + p1_ref[...] + b_ref[...], 0.0)
    o_ref[...] = jnp.dot(h, w_ref[...], preferred_element_type=jnp.float32)


def _relu_matmul(parts, b, w):
    return pl.pallas_call(
        _relu_mm_body,
        grid=(NB_G,),
        in_specs=[pl.BlockSpec((RB, D), lambda i: (i, 0)),
                  pl.BlockSpec((RB, D), lambda i: (i + NB_G, 0)),
                  pl.BlockSpec((1, D), lambda i: (0, 0)),
                  pl.BlockSpec((D, D), lambda i: (0, 0))],
        out_specs=pl.BlockSpec((RB, D), lambda i: (i, 0)),
        out_shape=jax.ShapeDtypeStruct((N_NODES, D), jnp.float32),
    )(parts, parts, b.reshape(1, D), w)


def _colsum_body(p0_ref, p1_ref, b_ref, o_ref, acc_ref):
    i = pl.program_id(0)

    @pl.when(i == 0)
    def _init():
        acc_ref[...] = jnp.zeros_like(acc_ref)

    h = jnp.maximum(p0_ref[...] + p1_ref[...] + b_ref[...], 0.0)
    acc_ref[...] += jnp.sum(h, axis=0, keepdims=True)

    @pl.when(i == pl.num_programs(0) - 1)
    def _fin():
        o_ref[...] = acc_ref[...]


def _colsum_relu(parts, b1):
    return pl.pallas_call(
        _colsum_body,
        grid=(NB_G,),
        in_specs=[pl.BlockSpec((RB, D), lambda i: (i, 0)),
                  pl.BlockSpec((RB, D), lambda i: (i + NB_G, 0)),
                  pl.BlockSpec((1, D), lambda i: (0, 0))],
        out_specs=pl.BlockSpec((1, D), lambda i: (0, 0)),
        out_shape=jax.ShapeDtypeStruct((1, D), jnp.float32),
        scratch_shapes=[pltpu.VMEM((1, D), jnp.float32)],
    )(parts, parts, b1.reshape(1, D))


def _bce_pos(z):
    return jnp.maximum(z, 0.0) - z + jnp.log(1.0 + jnp.exp(-jnp.abs(z)))


def _bce_neg(z):
    return jnp.maximum(z, 0.0) + jnp.log(1.0 + jnp.exp(-jnp.abs(z)))


def _loss_a_body(p0_ref, p1_ref, b_ref, cs_ref, bw_ref, neg_ref,
                 o_ref, acc_ref):
    i = pl.program_id(0)

    @pl.when(i == 0)
    def _init():
        acc_ref[0] = 0.0
        acc_ref[1] = 0.0

    c = 1.0 / (1.0 + jnp.exp(-cs_ref[...] / N_NODES))
    u = lax.dot_general(c, bw_ref[...], (((1,), (1,)), ((), ())),
                        preferred_element_type=jnp.float32)

    h = jnp.maximum(p0_ref[...] + p1_ref[...] + b_ref[...], 0.0)
    z = jnp.sum(h * u, axis=1)
    acc_ref[0] += jnp.sum(_bce_pos(z))

    pos_z = jnp.sum(h * h, axis=1)
    negs = jnp.sum(h[:, None, :] * neg_ref[...], axis=2)
    acc_ref[1] += jnp.sum(_bce_pos(pos_z)) + jnp.sum(_bce_neg(negs))

    @pl.when(i == pl.num_programs(0) - 1)
    def _fin():
        o_ref[0, 0] = acc_ref[0]
        o_ref[0, 1] = acc_ref[1]


def _loss_a(s2a, b1, colsum, bi_weights, negative_in):
    return pl.pallas_call(
        _loss_a_body,
        grid=(NB_G,),
        in_specs=[
            pl.BlockSpec((RB, D), lambda i: (i, 0)),
            pl.BlockSpec((RB, D), lambda i: (i + NB_G, 0)),
            pl.BlockSpec((1, D), lambda i: (0, 0)),
            pl.BlockSpec((1, D), lambda i: (0, 0)),
            pl.BlockSpec((D, D), lambda i: (0, 0)),
            pl.BlockSpec((RB, NEG, D), lambda i: (i, 0, 0)),
        ],
        out_specs=pl.BlockSpec(memory_space=pltpu.SMEM),
        out_shape=jax.ShapeDtypeStruct((1, 2), jnp.float32),
        scratch_shapes=[pltpu.SMEM((2,), jnp.float32)],
    )(s2a, s2a, b1.reshape(1, D), colsum, bi_weights, negative_in)


def _loss_b_body(p0_ref, p1_ref, b_ref, cs_ref, bw_ref, la_ref,
                 o_ref, acc_ref):
    i = pl.program_id(0)

    @pl.when(i == 0)
    def _init():
        acc_ref[0] = 0.0

    c = 1.0 / (1.0 + jnp.exp(-cs_ref[...] / N_NODES))
    u = lax.dot_general(c, bw_ref[...], (((1,), (1,)), ((), ())),
                        preferred_element_type=jnp.float32)

    h = jnp.maximum(p0_ref[...] + p1_ref[...] + b_ref[...], 0.0)
    z = jnp.sum(h * u, axis=1)
    acc_ref[0] += jnp.sum(_bce_neg(z))

    @pl.when(i == pl.num_programs(0) - 1)
    def _fin():
        dgi = (la_ref[0, 0] + acc_ref[0]) / (2 * N_NODES)
        o_ref[0, 0] = dgi + INS_LOSS_W * la_ref[0, 1] / N_NODES


def _loss_b(s2b, b1, colsum, bi_weights, la):
    return pl.pallas_call(
        _loss_b_body,
        grid=(NB_G,),
        in_specs=[
            pl.BlockSpec((RB, D), lambda i: (i, 0)),
            pl.BlockSpec((RB, D), lambda i: (i + NB_G, 0)),
            pl.BlockSpec((1, D), lambda i: (0, 0)),
            pl.BlockSpec((1, D), lambda i: (0, 0)),
            pl.BlockSpec((D, D), lambda i: (0, 0)),
            pl.BlockSpec(memory_space=pltpu.SMEM),
        ],
        out_specs=pl.BlockSpec(memory_space=pltpu.SMEM),
        out_shape=jax.ShapeDtypeStruct((1, 1), jnp.float32),
        scratch_shapes=[pltpu.SMEM((1,), jnp.float32)],
    )(s2b, s2b, b1.reshape(1, D), colsum, bi_weights, la)


def _edges_2d(ei):
    npad = E_PAD - N_EDGES
    pad_lanes = jnp.arange(npad, dtype=jnp.int32) % 128
    src = jnp.concatenate(
        [ei[0].astype(jnp.int32), pad_lanes]
    ).reshape(IDX_ROWS, EK)
    dst = jnp.concatenate(
        [ei[1].astype(jnp.int32), N_NODES + (pad_lanes % 240)]
    ).reshape(IDX_ROWS, EK)
    return src, dst


def kernel(x, edge_index, corp_x, corp_edge_index, negative_in,
           W0, b0, W1, b1, bi_weights):
    src_a, dst_a = _edges_2d(edge_index)
    src_b, dst_b = _edges_2d(corp_edge_index)
    zs = jnp.zeros((STRIPE, D), jnp.float32)

    y0a = _matmul(x, W0)
    y0b = _matmul(corp_x, W0)
    s1a = _sc_segsum(y0a, src_a, dst_a, zs)
    s1b = _sc_segsum(y0b, src_b, dst_b, zs)
    y1a = _relu_matmul(s1a, b0, W1)
    s2a = _sc_segsum(y1a, src_a, dst_a, zs)
    y1b = _relu_matmul(s1b, b0, W1)
    s2b = _sc_segsum(y1b, src_b, dst_b, zs)
    cs = _colsum_relu(s2a, b1)
    la = _loss_a(s2a, b1, cs, bi_weights, negative_in)
    out = _loss_b(s2b, b1, cs, bi_weights, la)
    return out.reshape(())


# speedup vs baseline: 1.0233x; 1.0233x over previous
import jax
import jax.numpy as jnp
from jax import lax
from jax.experimental import pallas as pl
from jax.experimental.pallas import tpu as pltpu
from jax.experimental.pallas import tpu_sc as plsc

N_NODES = 10000
N_EDGES = 320000
D = 128
NEG = 10
INS_LOSS_W = 1e-05

NC = 2
NS = 16
NW = NC * NS
ACC_ROWS = 10240
STRIPE = ACC_ROWS // NS
EK = 128
IB = 16
IDX_ROWS = 2560
E_PAD = IDX_ROWS * EK
NBLK = IDX_ROWS // IB


def _sc_segsum_body(y_hbm, src_hbm, dst_hbm, zeros_hbm, out_hbm,
                    acc_shared, rows_a, rows_b, src_blk, dst_blk,
                    src_blk2, dst_blk2,
                    gsem_a, gsem_b, ssem_a, ssem_b, isem_s, isem_d):
    c = lax.axis_index("c")
    s = lax.axis_index("s")
    w = c * NS + s

    row0 = s * STRIPE
    pi0 = pltpu.async_copy(src_hbm.at[pl.ds(w * IB, IB), :], src_blk, isem_s)
    pi1 = pltpu.async_copy(dst_hbm.at[pl.ds(w * IB, IB), :], dst_blk, isem_d)
    pltpu.sync_copy(zeros_hbm, acc_shared.at[pl.ds(row0, STRIPE), :])
    plsc.subcore_barrier()
    pi0.wait()
    pi1.wait()

    n_iter = NBLK // NW
    isets = ((src_blk, dst_blk), (src_blk2, dst_blk2))

    def _process(i, cur_set, nxt_set):
        src_c, dst_c = cur_set
        src_n, dst_n = nxt_set
        idesc = []

        @pl.when(i < n_iter - 1)
        def _prefetch():
            r1 = (w + (i + 1) * NW) * IB
            idesc.append(pltpu.async_copy(
                src_hbm.at[pl.ds(r1, IB), :], src_n, isem_s))
            idesc.append(pltpu.async_copy(
                dst_hbm.at[pl.ds(r1, IB), :], dst_n, isem_d))

        bufs = ((rows_a, gsem_a, ssem_a), (rows_b, gsem_b, ssem_b))
        gd = [None, None]
        sd = [None, None]

        @pl.when(i > 0)
        def _wait_prev_a():
            pltpu.make_async_copy(
                rows_a, acc_shared.at[dst_c.at[0]], ssem_a).wait()

        gd[0] = pltpu.async_copy(y_hbm.at[src_c.at[0]], rows_a, gsem_a)
        for j in range(IB):
            cur = j % 2
            nxt = (j + 1) % 2
            buf, _, ssem = bufs[cur]
            nbuf, ngsem, _ = bufs[nxt]
            if j + 1 < IB:
                if sd[nxt] is not None:
                    sd[nxt].wait()
                elif j == 0:
                    @pl.when(i > 0)
                    def _wait_prev_b():
                        pltpu.make_async_copy(
                            rows_b, acc_shared.at[dst_c.at[0]],
                            ssem_b).wait()
                gd[nxt] = pltpu.async_copy(
                    y_hbm.at[src_c.at[j + 1]], nbuf, ngsem)
            gd[cur].wait()
            sd[cur] = pltpu.async_copy(
                buf, acc_shared.at[dst_c.at[j]], ssem, add=True)

        @pl.when(i == n_iter - 1)
        def _drain_tail():
            sd[0].wait()
            sd[1].wait()

        @pl.when(i < n_iter - 1)
        def _wait_prefetch():
            pltpu.make_async_copy(
                src_hbm.at[pl.ds(0, IB), :], src_n, isem_s).wait()
            pltpu.make_async_copy(
                dst_hbm.at[pl.ds(0, IB), :], dst_n, isem_d).wait()

    def _block(i, _):
        @pl.when(i % 2 == 0)
        def _even():
            _process(i, isets[0], isets[1])

        @pl.when(i % 2 == 1)
        def _odd():
            _process(i, isets[1], isets[0])
        return ()
    lax.fori_loop(0, n_iter, _block, ())
    plsc.subcore_barrier()

    @pl.when(s < NS - 1)
    def _wr_full():
        pltpu.sync_copy(acc_shared.at[pl.ds(row0, STRIPE), :],
                        out_hbm.at[pl.ds(c * N_NODES + row0, STRIPE), :])

    @pl.when(s == NS - 1)
    def _wr_tail():
        tail = N_NODES - (NS - 1) * STRIPE
        base = (NS - 1) * STRIPE
        pltpu.sync_copy(acc_shared.at[pl.ds(base, tail), :],
                        out_hbm.at[pl.ds(c * N_NODES + base, tail), :])


def _sc_segsum(y, src2d, dst2d, zeros_stripe):
    mesh = plsc.VectorSubcoreMesh(core_axis_name="c", subcore_axis_name="s")
    return pl.kernel(
        _sc_segsum_body,
        out_type=jax.ShapeDtypeStruct((2 * N_NODES, D), jnp.float32),
        mesh=mesh,
        scratch_types=[
            pltpu.VMEM_SHARED((ACC_ROWS, D), jnp.float32),
            pltpu.VMEM((EK, D), jnp.float32),
            pltpu.VMEM((EK, D), jnp.float32),
            pltpu.VMEM((IB, EK), jnp.int32),
            pltpu.VMEM((IB, EK), jnp.int32),
            pltpu.VMEM((IB, EK), jnp.int32),
            pltpu.VMEM((IB, EK), jnp.int32),
            pltpu.SemaphoreType.DMA,
            pltpu.SemaphoreType.DMA,
            pltpu.SemaphoreType.DMA,
            pltpu.SemaphoreType.DMA,
            pltpu.SemaphoreType.DMA,
            pltpu.SemaphoreType.DMA,
        ],
    )(y, src2d, dst2d, zeros_stripe)


RB = 1000
NB_G = N_NODES // RB


def _mm_body(x_ref, w_ref, o_ref):
    o_ref[...] = jnp.dot(x_ref[...], w_ref[...],
                         preferred_element_type=jnp.float32)


def _matmul(x, w):
    return pl.pallas_call(
        _mm_body,
        grid=(NB_G,),
        in_specs=[pl.BlockSpec((RB, D), lambda i: (i, 0)),
                  pl.BlockSpec((D, D), lambda i: (0, 0))],
        out_specs=pl.BlockSpec((RB, D), lambda i: (i, 0)),
        out_shape=jax.ShapeDtypeStruct((N_NODES, D), jnp.float32),
    )(x, w)


def _relu_mm_body(p0_ref, p1_ref, b_ref, w_ref, o_ref):
    h = jnp.maximum(p0_ref[...] + p1_ref[...] + b_ref[...], 0.0)
    o_ref[...] = jnp.dot(h, w_ref[...], preferred_element_type=jnp.float32)


def _relu_matmul(parts, b, w):
    return pl.pallas_call(
        _relu_mm_body,
        grid=(NB_G,),
        in_specs=[pl.BlockSpec((RB, D), lambda i: (i, 0)),
                  pl.BlockSpec((RB, D), lambda i: (i + NB_G, 0)),
                  pl.BlockSpec((1, D), lambda i: (0, 0)),
                  pl.BlockSpec((D, D), lambda i: (0, 0))],
        out_specs=pl.BlockSpec((RB, D), lambda i: (i, 0)),
        out_shape=jax.ShapeDtypeStruct((N_NODES, D), jnp.float32),
    )(parts, parts, b.reshape(1, D), w)


def _colsum_body(p0_ref, p1_ref, b_ref, o_ref, acc_ref):
    i = pl.program_id(0)

    @pl.when(i == 0)
    def _init():
        acc_ref[...] = jnp.zeros_like(acc_ref)

    h = jnp.maximum(p0_ref[...] + p1_ref[...] + b_ref[...], 0.0)
    acc_ref[...] += jnp.sum(h, axis=0, keepdims=True)

    @pl.when(i == pl.num_programs(0) - 1)
    def _fin():
        o_ref[...] = acc_ref[...]


def _colsum_relu(parts, b1):
    return pl.pallas_call(
        _colsum_body,
        grid=(NB_G,),
        in_specs=[pl.BlockSpec((RB, D), lambda i: (i, 0)),
                  pl.BlockSpec((RB, D), lambda i: (i + NB_G, 0)),
                  pl.BlockSpec((1, D), lambda i: (0, 0))],
        out_specs=pl.BlockSpec((1, D), lambda i: (0, 0)),
        out_shape=jax.ShapeDtypeStruct((1, D), jnp.float32),
        scratch_shapes=[pltpu.VMEM((1, D), jnp.float32)],
    )(parts, parts, b1.reshape(1, D))


def _bce_pos(z):
    return jnp.maximum(z, 0.0) - z + jnp.log(1.0 + jnp.exp(-jnp.abs(z)))


def _bce_neg(z):
    return jnp.maximum(z, 0.0) + jnp.log(1.0 + jnp.exp(-jnp.abs(z)))


def _loss_a_body(p0_ref, p1_ref, b_ref, cs_ref, bw_ref, neg_ref,
                 o_ref, acc_ref):
    i = pl.program_id(0)

    @pl.when(i == 0)
    def _init():
        acc_ref[0] = 0.0
        acc_ref[1] = 0.0

    c = 1.0 / (1.0 + jnp.exp(-cs_ref[...] / N_NODES))
    u = lax.dot_general(c, bw_ref[...], (((1,), (1,)), ((), ())),
                        preferred_element_type=jnp.float32)

    h = jnp.maximum(p0_ref[...] + p1_ref[...] + b_ref[...], 0.0)
    z = jnp.sum(h * u, axis=1)
    acc_ref[0] += jnp.sum(_bce_pos(z))

    pos_z = jnp.sum(h * h, axis=1)
    negs = jnp.sum(h[:, None, :] * neg_ref[...], axis=2)
    acc_ref[1] += jnp.sum(_bce_pos(pos_z)) + jnp.sum(_bce_neg(negs))

    @pl.when(i == pl.num_programs(0) - 1)
    def _fin():
        o_ref[0, 0] = acc_ref[0]
        o_ref[0, 1] = acc_ref[1]


def _loss_a(s2a, b1, colsum, bi_weights, negative_in):
    return pl.pallas_call(
        _loss_a_body,
        grid=(NB_G,),
        in_specs=[
            pl.BlockSpec((RB, D), lambda i: (i, 0)),
            pl.BlockSpec((RB, D), lambda i: (i + NB_G, 0)),
            pl.BlockSpec((1, D), lambda i: (0, 0)),
            pl.BlockSpec((1, D), lambda i: (0, 0)),
            pl.BlockSpec((D, D), lambda i: (0, 0)),
            pl.BlockSpec((RB, NEG, D), lambda i: (i, 0, 0)),
        ],
        out_specs=pl.BlockSpec(memory_space=pltpu.SMEM),
        out_shape=jax.ShapeDtypeStruct((1, 2), jnp.float32),
        scratch_shapes=[pltpu.SMEM((2,), jnp.float32)],
    )(s2a, s2a, b1.reshape(1, D), colsum, bi_weights, negative_in)


def _loss_b_body(p0_ref, p1_ref, b_ref, cs_ref, bw_ref, la_ref,
                 o_ref, acc_ref):
    i = pl.program_id(0)

    @pl.when(i == 0)
    def _init():
        acc_ref[0] = 0.0

    c = 1.0 / (1.0 + jnp.exp(-cs_ref[...] / N_NODES))
    u = lax.dot_general(c, bw_ref[...], (((1,), (1,)), ((), ())),
                        preferred_element_type=jnp.float32)

    h = jnp.maximum(p0_ref[...] + p1_ref[...] + b_ref[...], 0.0)
    z = jnp.sum(h * u, axis=1)
    acc_ref[0] += jnp.sum(_bce_neg(z))

    @pl.when(i == pl.num_programs(0) - 1)
    def _fin():
        dgi = (la_ref[0, 0] + acc_ref[0]) / (2 * N_NODES)
        o_ref[0, 0] = dgi + INS_LOSS_W * la_ref[0, 1] / N_NODES


def _loss_b(s2b, b1, colsum, bi_weights, la):
    return pl.pallas_call(
        _loss_b_body,
        grid=(NB_G,),
        in_specs=[
            pl.BlockSpec((RB, D), lambda i: (i, 0)),
            pl.BlockSpec((RB, D), lambda i: (i + NB_G, 0)),
            pl.BlockSpec((1, D), lambda i: (0, 0)),
            pl.BlockSpec((1, D), lambda i: (0, 0)),
            pl.BlockSpec((D, D), lambda i: (0, 0)),
            pl.BlockSpec(memory_space=pltpu.SMEM),
        ],
        out_specs=pl.BlockSpec(memory_space=pltpu.SMEM),
        out_shape=jax.ShapeDtypeStruct((1, 1), jnp.float32),
        scratch_shapes=[pltpu.SMEM((1,), jnp.float32)],
    )(s2b, s2b, b1.reshape(1, D), colsum, bi_weights, la)


def _edges_2d(ei):
    npad = E_PAD - N_EDGES
    pad_lanes = jnp.arange(npad, dtype=jnp.int32) % 128
    src = jnp.concatenate(
        [ei[0].astype(jnp.int32), pad_lanes]
    ).reshape(IDX_ROWS, EK)
    dst = jnp.concatenate(
        [ei[1].astype(jnp.int32), N_NODES + (pad_lanes % 240)]
    ).reshape(IDX_ROWS, EK)
    return src, dst


def kernel(x, edge_index, corp_x, corp_edge_index, negative_in,
           W0, b0, W1, b1, bi_weights):
    src_a, dst_a = _edges_2d(edge_index)
    src_b, dst_b = _edges_2d(corp_edge_index)
    zs = jnp.zeros((STRIPE, D), jnp.float32)

    y0a = _matmul(x, W0)
    y0b = _matmul(corp_x, W0)
    s1a = _sc_segsum(y0a, src_a, dst_a, zs)
    s1b = _sc_segsum(y0b, src_b, dst_b, zs)
    y1a = _relu_matmul(s1a, b0, W1)
    s2a = _sc_segsum(y1a, src_a, dst_a, zs)
    y1b = _relu_matmul(s1b, b0, W1)
    s2b = _sc_segsum(y1b, src_b, dst_b, zs)
    cs = _colsum_relu(s2a, b1)
    la = _loss_a(s2a, b1, cs, bi_weights, negative_in)
    out = _loss_b(s2b, b1, cs, bi_weights, la)
    return out.reshape(())
